# baseline (speedup 1.0000x reference)
import functools

import jax
import jax.numpy as jnp
from jax import lax
from jax.experimental import pallas as pl
from jax.experimental.pallas import tpu as pltpu
from jax.experimental.pallas import tpu_sc as plsc

_B = 16384
_D = 32
_PD = 128
_NC = 2
_NS = 16
_NW = _NC * _NS
_BPW = _B // _NW
_CHUNK = 128
_NCHUNK = _BPW // _CHUNK

_TC_BLOCK = 2048


def _sc_gather_body(uidx_hbm, iidx_hbm, utab, itab,
                    u_out, i_out,
                    uidx_v, iidx_v, ubuf, ibuf, sem):
    wid = lax.axis_index("s") * _NC + lax.axis_index("c")
    base = wid * _BPW
    pltpu.sync_copy(uidx_hbm.at[pl.ds(wid * _NCHUNK, _NCHUNK)], uidx_v)
    pltpu.sync_copy(iidx_hbm.at[pl.ds(wid * _NCHUNK, _NCHUNK)], iidx_v)
    half = _NCHUNK // 2
    for p in range(2):
        copies = []
        for c in range(half):
            copies.append(pltpu.async_copy(
                utab.at[uidx_v.at[p * half + c]],
                ubuf.at[pl.ds(c * _CHUNK, _CHUNK)], sem))
            copies.append(pltpu.async_copy(
                itab.at[iidx_v.at[p * half + c]],
                ibuf.at[pl.ds(c * _CHUNK, _CHUNK)], sem))
        for cp in copies:
            cp.wait()
        pltpu.sync_copy(ubuf, u_out.at[pl.ds(base + p * half * _CHUNK,
                                             half * _CHUNK)])
        pltpu.sync_copy(ibuf, i_out.at[pl.ds(base + p * half * _CHUNK,
                                             half * _CHUNK)])


def _sc_gather(user_idx, item_idx, utab_pad, itab_pad):
    mesh = plsc.VectorSubcoreMesh(core_axis_name="c", subcore_axis_name="s")
    f = pl.kernel(
        _sc_gather_body,
        mesh=mesh,
        out_type=(
            jax.ShapeDtypeStruct((_B, _PD), jnp.float32),
            jax.ShapeDtypeStruct((_B, _PD), jnp.float32),
        ),
        scratch_types=[
            pltpu.VMEM((_NCHUNK, _CHUNK), jnp.int32),
            pltpu.VMEM((_NCHUNK, _CHUNK), jnp.int32),
            pltpu.VMEM((_BPW // 2, _PD), jnp.float32),
            pltpu.VMEM((_BPW // 2, _PD), jnp.float32),
            pltpu.SemaphoreType.DMA,
        ],
    )
    uidx2 = user_idx.reshape(_NW * _NCHUNK, _CHUNK)
    iidx2 = item_idx.reshape(_NW * _NCHUNK, _CHUNK)
    return f(uidx2, iidx2, utab_pad, itab_pad)


_RELAYOUT_BLK = 512


def _pad_body(tT_ref, out_ref):
    x = tT_ref[...]
    xt = x.T
    zeros = jnp.zeros((xt.shape[0], _PD - _D), dtype=xt.dtype)
    out_ref[...] = jnp.concatenate([xt, zeros], axis=1)


def _tc_pad(tabT):
    V = tabT.shape[1]
    grid = (V + _RELAYOUT_BLK - 1) // _RELAYOUT_BLK
    return pl.pallas_call(
        _pad_body,
        grid=(grid,),
        in_specs=[pl.BlockSpec((_D, _RELAYOUT_BLK), lambda b: (0, b))],
        out_specs=pl.BlockSpec((_RELAYOUT_BLK, _PD), lambda b: (b, 0)),
        out_shape=jax.ShapeDtypeStruct((V, _PD), jnp.float32),
    )(tabT)


def _tc_body(u_ref, i_ref, dn_ref, Wd_ref, bd_ref,
             W1u_ref, W1i_ref, W1d_ref, b1_ref,
             W2_ref, b2_ref, W3_ref, b3_ref, out_ref):
    u = u_ref[:, :_D]
    it = i_ref[:, :_D]
    dn = dn_ref[...]
    d = jnp.dot(dn, Wd_ref[...], preferred_element_type=jnp.float32) + bd_ref[...]
    s = u + it + d
    fm = 0.5 * jnp.sum(s * s - u * u - it * it - d * d, axis=1, keepdims=True)
    h = (jnp.dot(u, W1u_ref[...], preferred_element_type=jnp.float32)
         + jnp.dot(it, W1i_ref[...], preferred_element_type=jnp.float32)
         + jnp.dot(dn, W1d_ref[...], preferred_element_type=jnp.float32)
         + b1_ref[...])
    h = jnp.maximum(h, 0.0)
    h = jnp.maximum(
        jnp.dot(h, W2_ref[...], preferred_element_type=jnp.float32) + b2_ref[...],
        0.0)
    out = jnp.dot(h, W3_ref[...], preferred_element_type=jnp.float32) + b3_ref[...]
    out_ref[...] = out + fm


def _tc_compute(u, i, dense, Wd, bd, W1, b1, W2, b2, W3, b3):
    nd = dense.shape[1]
    h1 = W1.shape[1]
    h2 = W2.shape[1]
    W1u = W1[:_D]
    W1i = W1[_D:2 * _D]
    W1d = W1[2 * _D:]
    grid = _B // _TC_BLOCK

    def batch_spec(cols):
        return pl.BlockSpec((_TC_BLOCK, cols), lambda b: (b, 0))

    def full_spec(shape):
        return pl.BlockSpec(shape, lambda b: (0,) * len(shape))

    out = pl.pallas_call(
        _tc_body,
        grid=(grid,),
        in_specs=[
            batch_spec(_PD), batch_spec(_PD), batch_spec(nd),
            full_spec(Wd.shape), full_spec((1, _D)),
            full_spec(W1u.shape), full_spec(W1i.shape), full_spec(W1d.shape),
            full_spec((1, h1)),
            full_spec(W2.shape), full_spec((1, h2)),
            full_spec(W3.shape), full_spec((1, 1)),
        ],
        out_specs=pl.BlockSpec((_TC_BLOCK, 1), lambda b: (b, 0)),
        out_shape=jax.ShapeDtypeStruct((_B, 1), jnp.float32),
    )(u, i, dense, Wd, bd.reshape(1, _D),
      W1u, W1i, W1d, b1.reshape(1, h1),
      W2, b2.reshape(1, h2), W3, b3.reshape(1, 1))
    return out[:, 0]


def kernel(user_idx, item_idx, dense, user_emb, item_emb,
           Wd, bd, W1, b1, W2, b2, W3, b3):
    utab_pad = _tc_pad(user_emb.T)
    itab_pad = _tc_pad(item_emb.T)
    u, i = _sc_gather(user_idx.astype(jnp.int32), item_idx.astype(jnp.int32),
                      utab_pad, itab_pad)
    return _tc_compute(u, i, dense, Wd, bd, W1, b1, W2, b2, W3, b3)


# speedup vs baseline: 2.5539x; 2.5539x over previous
import functools

import jax
import jax.numpy as jnp
from jax import lax
from jax.experimental import pallas as pl
from jax.experimental.pallas import tpu as pltpu
from jax.experimental.pallas import tpu_sc as plsc

_B = 16384
_D = 32
_PD = 128
_NC = 2
_NS = 16
_NW = _NC * _NS
_BPW = _B // _NW
_CHUNK = 128
_NCHUNK = _BPW // _CHUNK

_TC_BLOCK = 2048


def _sc_gather_body(uidx_hbm, iidx_hbm, utab, itab,
                    u_out, i_out,
                    uidx_v, iidx_v, ubuf, ibuf, sem):
    wid = lax.axis_index("s") * _NC + lax.axis_index("c")
    base = wid * _BPW
    pltpu.sync_copy(uidx_hbm.at[pl.ds(wid * _NCHUNK, _NCHUNK)], uidx_v)
    pltpu.sync_copy(iidx_hbm.at[pl.ds(wid * _NCHUNK, _NCHUNK)], iidx_v)
    half = _NCHUNK // 2
    for p in range(2):
        copies = []
        for c in range(half):
            copies.append(pltpu.async_copy(
                utab.at[uidx_v.at[p * half + c]],
                ubuf.at[pl.ds(c * _CHUNK, _CHUNK)], sem))
            copies.append(pltpu.async_copy(
                itab.at[iidx_v.at[p * half + c]],
                ibuf.at[pl.ds(c * _CHUNK, _CHUNK)], sem))
        for cp in copies:
            cp.wait()
        pltpu.sync_copy(ubuf, u_out.at[pl.ds(base + p * half * _CHUNK,
                                             half * _CHUNK)])
        pltpu.sync_copy(ibuf, i_out.at[pl.ds(base + p * half * _CHUNK,
                                             half * _CHUNK)])


def _sc_gather(user_idx, item_idx, utab_pad, itab_pad):
    mesh = plsc.VectorSubcoreMesh(core_axis_name="c", subcore_axis_name="s")
    f = pl.kernel(
        _sc_gather_body,
        mesh=mesh,
        out_type=(
            jax.ShapeDtypeStruct((_B, _PD), jnp.float32),
            jax.ShapeDtypeStruct((_B, _PD), jnp.float32),
        ),
        scratch_types=[
            pltpu.VMEM((_NCHUNK, _CHUNK), jnp.int32),
            pltpu.VMEM((_NCHUNK, _CHUNK), jnp.int32),
            pltpu.VMEM((_BPW // 2, _PD), jnp.float32),
            pltpu.VMEM((_BPW // 2, _PD), jnp.float32),
            pltpu.SemaphoreType.DMA,
        ],
    )
    uidx2 = user_idx.reshape(_NW * _NCHUNK, _CHUNK)
    iidx2 = item_idx.reshape(_NW * _NCHUNK, _CHUNK)
    return f(uidx2, iidx2, utab_pad, itab_pad)


def _tc_body(u_ref, i_ref, dn_ref, Wd_ref, bd_ref,
             W1u_ref, W1i_ref, W1d_ref, b1_ref,
             W2_ref, b2_ref, W3_ref, b3_ref, out_ref):
    u = u_ref[:, :_D]
    it = i_ref[:, :_D]
    dn = dn_ref[...]
    d = jnp.dot(dn, Wd_ref[...], preferred_element_type=jnp.float32) + bd_ref[...]
    s = u + it + d
    fm = 0.5 * jnp.sum(s * s - u * u - it * it - d * d, axis=1, keepdims=True)
    h = (jnp.dot(u, W1u_ref[...], preferred_element_type=jnp.float32)
         + jnp.dot(it, W1i_ref[...], preferred_element_type=jnp.float32)
         + jnp.dot(dn, W1d_ref[...], preferred_element_type=jnp.float32)
         + b1_ref[...])
    h = jnp.maximum(h, 0.0)
    h = jnp.maximum(
        jnp.dot(h, W2_ref[...], preferred_element_type=jnp.float32) + b2_ref[...],
        0.0)
    out = jnp.dot(h, W3_ref[...], preferred_element_type=jnp.float32) + b3_ref[...]
    out_ref[...] = out + fm


def _tc_compute(u, i, dense, Wd, bd, W1, b1, W2, b2, W3, b3):
    nd = dense.shape[1]
    h1 = W1.shape[1]
    h2 = W2.shape[1]
    W1u = W1[:_D]
    W1i = W1[_D:2 * _D]
    W1d = W1[2 * _D:]
    grid = _B // _TC_BLOCK

    def batch_spec(cols):
        return pl.BlockSpec((_TC_BLOCK, cols), lambda b: (b, 0))

    def full_spec(shape):
        return pl.BlockSpec(shape, lambda b: (0,) * len(shape))

    out = pl.pallas_call(
        _tc_body,
        grid=(grid,),
        in_specs=[
            batch_spec(_PD), batch_spec(_PD), batch_spec(nd),
            full_spec(Wd.shape), full_spec((1, _D)),
            full_spec(W1u.shape), full_spec(W1i.shape), full_spec(W1d.shape),
            full_spec((1, h1)),
            full_spec(W2.shape), full_spec((1, h2)),
            full_spec(W3.shape), full_spec((1, 1)),
        ],
        out_specs=pl.BlockSpec((_TC_BLOCK, 1), lambda b: (b, 0)),
        out_shape=jax.ShapeDtypeStruct((_B, 1), jnp.float32),
    )(u, i, dense, Wd, bd.reshape(1, _D),
      W1u, W1i, W1d, b1.reshape(1, h1),
      W2, b2.reshape(1, h2), W3, b3.reshape(1, 1))
    return out[:, 0]


def kernel(user_idx, item_idx, dense, user_emb, item_emb,
           Wd, bd, W1, b1, W2, b2, W3, b3):
    utab_pad = jnp.pad(user_emb, ((0, 0), (0, _PD - _D)))
    itab_pad = jnp.pad(item_emb, ((0, 0), (0, _PD - _D)))
    u, i = _sc_gather(user_idx.astype(jnp.int32), item_idx.astype(jnp.int32),
                      utab_pad, itab_pad)
    return _tc_compute(u, i, dense, Wd, bd, W1, b1, W2, b2, W3, b3)
